# baseline (speedup 1.0000x reference)
import jax
import jax.numpy as jnp
from jax import lax
from jax.experimental import pallas as pl
from jax.experimental.pallas import tpu as pltpu
from jax.experimental.pallas import tpu_sc as plsc

BATCH = 4096
VOCAB = 100000
SEQ = 50
DIM = 64
NC = 2
NS = 16
NW = NC * NS
BPW = BATCH // NW
NBUF = 8
INFLIGHT = 4
LANES = 16
SEQP = 56


def _sc_body(idx_hbm, table_hbm, out_hbm, idx_v, rows_v, out_v, *sems):
    wid = lax.axis_index("s") * NC + lax.axis_index("c")
    pltpu.sync_copy(idx_hbm.at[pl.ds(wid * (BPW * SEQP), BPW * SEQP)], idx_v)

    def issue(r, b):
        pltpu.async_copy(table_hbm.at[idx_v.at[pl.ds(r * SEQP, SEQ)]],
                         rows_v.at[b], sems[b])

    def consume(r, b):
        pltpu.make_async_copy(table_hbm.at[idx_v.at[pl.ds(r * SEQP, SEQ)]],
                              rows_v.at[b], sems[b]).wait()
        rb = rows_v.at[b]
        for c in range(DIM // LANES):
            col = pl.ds(c * LANES, LANES)
            s0 = rb[0, col]
            s1 = rb[1, col]
            for k in range(2, SEQ, 2):
                s0 += rb[k, col]
                s1 += rb[k + 1, col]
            out_v[pl.ds(r * DIM + c * LANES, LANES)] = (s0 + s1) * (1.0 / SEQ)

    for r0 in range(INFLIGHT):
        issue(r0, r0)

    groups = BPW // NBUF

    def group(g, issue_next):
        for b in range(NBUF):
            r = g * NBUF + b
            consume(r, b)
            if issue_next:
                issue(r + INFLIGHT, (b + INFLIGHT) % NBUF)

    def steady(g, carry):
        group(g, True)
        return carry

    lax.fori_loop(0, groups - 1, steady, 0)

    def last_group():
        for b in range(NBUF):
            r = (groups - 1) * NBUF + b
            if b < INFLIGHT:
                issue(r + INFLIGHT, (b + INFLIGHT) % NBUF)
            consume(r, b)

    last_group()

    pltpu.sync_copy(out_v, out_hbm.at[pl.ds(wid * (BPW * DIM), BPW * DIM)])


_run = pl.kernel(
    _sc_body,
    out_type=jax.ShapeDtypeStruct((BATCH * DIM,), jnp.float32),
    mesh=plsc.VectorSubcoreMesh(core_axis_name="c", subcore_axis_name="s",
                                num_cores=NC, num_subcores=NS),
    scratch_types=[
        pltpu.VMEM((BPW * SEQP,), jnp.int32),
        pltpu.VMEM((NBUF, SEQ, DIM), jnp.float32),
        pltpu.VMEM((BPW * DIM,), jnp.float32),
    ] + [pltpu.SemaphoreType.DMA] * NBUF,
    compiler_params=pltpu.CompilerParams(use_tc_tiling_on_sc=False),
)


def kernel(input_seq_batch, table):
    idx = jnp.pad(input_seq_batch.astype(jnp.int32),
                  ((0, 0), (0, SEQP - SEQ))).reshape(BATCH * SEQP)
    return _run(idx, table).reshape(BATCH, DIM)


# speedup vs baseline: 1.1613x; 1.1613x over previous
import jax
import jax.numpy as jnp
from jax import lax
from jax.experimental import pallas as pl
from jax.experimental.pallas import tpu as pltpu
from jax.experimental.pallas import tpu_sc as plsc

BATCH = 4096
VOCAB = 100000
SEQ = 50
DIM = 64
NC = 2
NS = 16
NW = NC * NS
BPW = BATCH // NW
NBUF = 4
AHEAD = 3
LANES = 16
SEQP = 56


def _sc_body(idx_hbm, table_hbm, out_hbm, idx_v, rows_v, out_v, *sems):
    wid = lax.axis_index("s") * NC + lax.axis_index("c")
    pltpu.sync_copy(idx_hbm.at[pl.ds(wid * (BPW * SEQP), BPW * SEQP)], idx_v)

    def issue(r, b):
        pltpu.async_copy(table_hbm.at[idx_v.at[pl.ds(r * SEQP, SEQ)]],
                         rows_v.at[b], sems[b])

    def consume(r, b):
        pltpu.make_async_copy(table_hbm.at[idx_v.at[pl.ds(r * SEQP, SEQ)]],
                              rows_v.at[b], sems[b]).wait()
        rb = rows_v.at[b]
        for c in range(DIM // LANES):
            col = pl.ds(c * LANES, LANES)
            s0 = rb[0, col]
            s1 = rb[1, col]
            for k in range(2, SEQ, 2):
                s0 += rb[k, col]
                s1 += rb[k + 1, col]
            out_v[pl.ds(r * DIM + c * LANES, LANES)] = (s0 + s1) * (1.0 / SEQ)

    for r0 in range(AHEAD):
        issue(r0, r0)

    groups = BPW // NBUF

    def step(r, b):
        issue(r + AHEAD, (b + AHEAD) % NBUF)
        consume(r, b)

    def group(g, carry):
        for b in range(NBUF):
            step(g * NBUF + b, b)
        return carry

    lax.fori_loop(0, groups - 1, group, 0)

    for b in range(NBUF):
        r = (groups - 1) * NBUF + b
        if b == 0:
            issue(r + AHEAD, (b + AHEAD) % NBUF)
        consume(r, b)

    pltpu.sync_copy(out_v, out_hbm.at[pl.ds(wid * (BPW * DIM), BPW * DIM)])


_run = pl.kernel(
    _sc_body,
    out_type=jax.ShapeDtypeStruct((BATCH * DIM,), jnp.float32),
    mesh=plsc.VectorSubcoreMesh(core_axis_name="c", subcore_axis_name="s",
                                num_cores=NC, num_subcores=NS),
    scratch_types=[
        pltpu.VMEM((BPW * SEQP,), jnp.int32),
        pltpu.VMEM((NBUF, SEQ, DIM), jnp.float32),
        pltpu.VMEM((BPW * DIM,), jnp.float32),
    ] + [pltpu.SemaphoreType.DMA] * NBUF,
    compiler_params=pltpu.CompilerParams(use_tc_tiling_on_sc=False),
)


def kernel(input_seq_batch, table):
    idx = jnp.pad(input_seq_batch.astype(jnp.int32),
                  ((0, 0), (0, SEQP - SEQ))).reshape(BATCH * SEQP)
    return _run(idx, table).reshape(BATCH, DIM)
